# Initial kernel scaffold; baseline (speedup 1.0000x reference)
import jax
import jax.numpy as jnp
from jax.experimental import pallas as pl


def kernel(pc_source, pred_flow):
    raise NotImplementedError("write your pallas kernel here")


# speedup vs baseline: 23.0891x; 23.0891x over previous
import functools

import jax
import jax.numpy as jnp
import numpy as np
from jax.experimental import pallas as pl

_K = 16
_RADIUS2 = np.float32(0.1 * 0.1)
_N = 8192
_S = 4
_R = 256
_CB = 256


def _dot(a, b):
    return jax.lax.dot_general(a, b, (((1,), (0,)), ((), ())),
                               preferred_element_type=jnp.float32)


def _ball_loss_kernel(pc_ref, pcT_ref, flow_ref, flowT_ref, out_ref):
    q = pc_ref[...]
    qq = jnp.sum(q * q, axis=1, keepdims=True)
    frows = [flow_ref[s] for s in range(_S)]
    fqq = [jnp.sum(f * f, axis=1, keepdims=True) for f in frows]

    ra = jax.lax.broadcasted_iota(jnp.int32, (_CB, _CB), 0)
    ca = jax.lax.broadcasted_iota(jnp.int32, (_CB, _CB), 1)
    tril = (ra < ca).astype(jnp.bfloat16)

    def chunk(cidx, carry):
        cnt, accs, fsts = carry
        c0 = cidx * _CB
        sT = pcT_ref[:, pl.ds(c0, _CB)]
        ss = jnp.sum(sT * sT, axis=0, keepdims=True)
        d2 = qq + ss - 2.0 * _dot(q, sT)
        mask = d2 < _RADIUS2
        maskb = mask.astype(jnp.bfloat16)
        prefx = _dot(maskb, tril)
        ranks = cnt + prefx
        sel = mask & (ranks < float(_K))
        fst = mask & (ranks == 0.0)
        new_cnt = cnt + jnp.sum(mask.astype(jnp.float32), axis=1, keepdims=True)
        new_accs = []
        new_fsts = []
        for s in range(_S):
            fT = flowT_ref[s, :, pl.ds(c0, _CB)]
            fss = jnp.sum(fT * fT, axis=0, keepdims=True)
            fd2 = fqq[s] + fss - 2.0 * _dot(frows[s], fT)
            n = jnp.sqrt(jnp.maximum(fd2, 0.0))
            new_accs.append(accs[s] + jnp.sum(jnp.where(sel, n, 0.0),
                                              axis=1, keepdims=True))
            new_fsts.append(fsts[s] + jnp.sum(jnp.where(fst, n, 0.0),
                                              axis=1, keepdims=True))
        return new_cnt, tuple(new_accs), tuple(new_fsts)

    zeros = jnp.zeros((_R, 1), jnp.float32)
    init = (zeros, (zeros,) * _S, (zeros,) * _S)
    cnt, accs, fsts = jax.lax.fori_loop(0, _N // _CB, chunk, init)

    pad = float(_K) - jnp.minimum(cnt, float(_K))
    total = sum(accs[s] + pad * fsts[s] for s in range(_S))
    partial = (jnp.sum(total, axis=0, keepdims=True)
               * np.float32(1.0 / (_K * _S * _N)))

    @pl.when(pl.program_id(0) == 0)
    def _init():
        out_ref[...] = partial

    @pl.when(pl.program_id(0) != 0)
    def _acc():
        out_ref[...] += partial


def kernel(pc_source, pred_flow):
    pc = pc_source[0]
    pc8 = jnp.pad(pc, ((0, 0), (0, 5)))
    flow8 = jnp.pad(pred_flow, ((0, 0), (0, 0), (0, 5)))
    pcT = jnp.transpose(pc8, (1, 0))
    flowT = jnp.transpose(flow8, (0, 2, 1))

    out = pl.pallas_call(
        _ball_loss_kernel,
        grid=(_N // _R,),
        in_specs=[
            pl.BlockSpec((_R, 8), lambda i: (i, 0)),
            pl.BlockSpec((8, _N), lambda i: (0, 0)),
            pl.BlockSpec((_S, _R, 8), lambda i: (0, i, 0)),
            pl.BlockSpec((_S, 8, _N), lambda i: (0, 0, 0)),
        ],
        out_specs=pl.BlockSpec((1, 1), lambda i: (0, 0)),
        out_shape=jax.ShapeDtypeStruct((1, 1), jnp.float32),
    )(pc8, pcT, flow8, flowT)
    return out[0, 0]


# speedup vs baseline: 33.9160x; 1.4689x over previous
import jax
import jax.numpy as jnp
import numpy as np
from jax.experimental import pallas as pl

_K = 16
_RADIUS2 = np.float32(0.1 * 0.1)
_N = 8192
_S = 4
_R = 256
_CB = 256


def _dot(a, b):
    return jax.lax.dot_general(a, b, (((1,), (0,)), ((), ())),
                               preferred_element_type=jnp.float32)


def _ball_loss_kernel(pcr_ref, pcc_ref, fr_ref, fc_ref, out_ref):
    q = pcr_ref[...]
    frows = [fr_ref[s] for s in range(_S)]

    ra = jax.lax.broadcasted_iota(jnp.int32, (_CB, _CB), 0)
    ca = jax.lax.broadcasted_iota(jnp.int32, (_CB, _CB), 1)
    tril = (ra < ca).astype(jnp.bfloat16)

    def chunk(cidx, carry):
        cnt, acc, fstn = carry
        c0 = cidx * _CB
        d2 = _dot(q, pcc_ref[:, pl.ds(c0, _CB)])
        mask = d2 < _RADIUS2
        prefx = _dot(mask.astype(jnp.bfloat16), tril)
        ranks = cnt + prefx
        sel = mask & (ranks < float(_K))
        fst = mask & (ranks == 0.0)
        nsum = None
        for s in range(_S):
            fd2 = _dot(frows[s], fc_ref[s, :, pl.ds(c0, _CB)])
            n = jnp.sqrt(jnp.maximum(fd2, 0.0))
            nsum = n if nsum is None else nsum + n
        acc = acc + jnp.sum(jnp.where(sel, nsum, 0.0), axis=1, keepdims=True)
        fstn = fstn + jnp.sum(jnp.where(fst, nsum, 0.0), axis=1, keepdims=True)
        cnt = cnt + jnp.sum(mask.astype(jnp.float32), axis=1, keepdims=True)
        return cnt, acc, fstn

    zeros = jnp.zeros((_R, 1), jnp.float32)
    cnt, acc, fstn = jax.lax.fori_loop(0, _N // _CB, chunk, (zeros,) * 3)

    pad = float(_K) - jnp.minimum(cnt, float(_K))
    total = acc + pad * fstn
    partial = (jnp.sum(total, axis=0, keepdims=True)
               * np.float32(1.0 / (_K * _S * _N)))

    @pl.when(pl.program_id(0) == 0)
    def _init():
        out_ref[...] = partial

    @pl.when(pl.program_id(0) != 0)
    def _acc():
        out_ref[...] += partial


def _row_op(x):
    sq = jnp.sum(x * x, axis=-1, keepdims=True)
    one = jnp.ones_like(sq)
    zero = jnp.zeros((x.shape[0], 3), x.dtype)
    return jnp.concatenate([x, sq, one, zero], axis=-1)


def _col_op(x):
    sq = jnp.sum(x * x, axis=-1, keepdims=True)
    one = jnp.ones_like(sq)
    zero = jnp.zeros((x.shape[0], 3), x.dtype)
    return jnp.concatenate([-2.0 * x, one, sq, zero], axis=-1).T


def kernel(pc_source, pred_flow):
    pc = pc_source[0]
    pcr = _row_op(pc)
    pcc = _col_op(pc)
    fr = jax.vmap(_row_op)(pred_flow)
    fc = jax.vmap(_col_op)(pred_flow)

    out = pl.pallas_call(
        _ball_loss_kernel,
        grid=(_N // _R,),
        in_specs=[
            pl.BlockSpec((_R, 8), lambda i: (i, 0)),
            pl.BlockSpec((8, _N), lambda i: (0, 0)),
            pl.BlockSpec((_S, _R, 8), lambda i: (0, i, 0)),
            pl.BlockSpec((_S, 8, _N), lambda i: (0, 0, 0)),
        ],
        out_specs=pl.BlockSpec((1, 1), lambda i: (0, 0)),
        out_shape=jax.ShapeDtypeStruct((1, 1), jnp.float32),
    )(pcr, pcc, fr, fc)
    return out[0, 0]


# speedup vs baseline: 41.5958x; 1.2264x over previous
import jax
import jax.numpy as jnp
import numpy as np
from jax.experimental import pallas as pl

_K = 16
_RADIUS2 = np.float32(0.1 * 0.1)
_N = 8192
_S = 4
_R = 256
_CB = 256


def _dot(a, b):
    return jax.lax.dot_general(a, b, (((1,), (0,)), ((), ())),
                               preferred_element_type=jnp.float32)


def _ball_loss_kernel(pcr_ref, pcc_ref, fr_ref, fc_ref, out_ref):
    q = pcr_ref[...]
    frows = [fr_ref[s] for s in range(_S)]

    ra = jax.lax.broadcasted_iota(jnp.int32, (_CB, _CB), 0)
    ca = jax.lax.broadcasted_iota(jnp.int32, (_CB, _CB), 1)
    tril = (ra < ca).astype(jnp.bfloat16)

    def chunk(cidx, carry):
        cnt, acc, fstn = carry
        c0 = cidx * _CB
        d2 = _dot(q, pcc_ref[:, pl.ds(c0, _CB)])
        mask = d2 < _RADIUS2
        prefx = _dot(mask.astype(jnp.bfloat16), tril)
        ranks = cnt + prefx
        sel = mask & (ranks < float(_K))
        fst = mask & (ranks == 0.0)
        nsum = None
        for s in range(_S):
            fd2 = _dot(frows[s], fc_ref[s, :, pl.ds(c0, _CB)])
            n = fd2 * jax.lax.rsqrt(jnp.maximum(fd2, 1e-12))
            nsum = n if nsum is None else nsum + n
        acc = acc + jnp.sum(jnp.where(sel, nsum, 0.0), axis=1, keepdims=True)
        fstn = fstn + jnp.sum(jnp.where(fst, nsum, 0.0), axis=1, keepdims=True)
        cnt = cnt + jnp.sum(mask.astype(jnp.float32), axis=1, keepdims=True)
        return cnt, acc, fstn

    zeros = jnp.zeros((_R, 1), jnp.float32)
    cnt, acc, fstn = jax.lax.fori_loop(0, _N // _CB, chunk, (zeros,) * 3)

    pad = float(_K) - jnp.minimum(cnt, float(_K))
    total = acc + pad * fstn
    partial = (jnp.sum(total, axis=0, keepdims=True)
               * np.float32(1.0 / (_K * _S * _N)))

    @pl.when(pl.program_id(0) == 0)
    def _init():
        out_ref[...] = partial

    @pl.when(pl.program_id(0) != 0)
    def _acc():
        out_ref[...] += partial


def _row_op(x):
    sq = jnp.sum(x * x, axis=-1, keepdims=True)
    one = jnp.ones_like(sq)
    zero = jnp.zeros((x.shape[0], 3), x.dtype)
    return jnp.concatenate([x, sq, one, zero], axis=-1)


def _col_op(x):
    sq = jnp.sum(x * x, axis=-1, keepdims=True)
    one = jnp.ones_like(sq)
    zero = jnp.zeros((x.shape[0], 3), x.dtype)
    return jnp.concatenate([-2.0 * x, one, sq, zero], axis=-1).T


def kernel(pc_source, pred_flow):
    pc = pc_source[0]
    pcr = _row_op(pc)
    pcc = _col_op(pc)
    fr = jax.vmap(_row_op)(pred_flow)
    fc = jax.vmap(_col_op)(pred_flow)

    out = pl.pallas_call(
        _ball_loss_kernel,
        grid=(_N // _R,),
        in_specs=[
            pl.BlockSpec((_R, 8), lambda i: (i, 0)),
            pl.BlockSpec((8, _N), lambda i: (0, 0)),
            pl.BlockSpec((_S, _R, 8), lambda i: (0, i, 0)),
            pl.BlockSpec((_S, 8, _N), lambda i: (0, 0, 0)),
        ],
        out_specs=pl.BlockSpec((1, 1), lambda i: (0, 0)),
        out_shape=jax.ShapeDtypeStruct((1, 1), jnp.float32),
    )(pcr, pcc, fr, fc)
    return out[0, 0]
